# baseline (speedup 1.0000x reference)
import functools

import jax
import jax.numpy as jnp
from jax import lax
from jax.experimental import pallas as pl
from jax.experimental.pallas import tpu as pltpu
from jax.experimental.pallas import tpu_sc as plsc

F = 26
V = 100000
D = 32
VP = V // 4
NC = 2
NS = 16
NW = NC * NS
CHUNK = 32
VB = 1000


RB = 1000


def _repack_body(eye_ref, p4_ref, x_ref, o_ref, zbuf):
    b = pl.program_id(1)

    @pl.when(b == 0)
    def _load():
        zbuf[...] = lax.dot_general(
            x_ref[0], eye_ref[...], (((0,), (0,)), ((), ())),
            preferred_element_type=jnp.float32)

    r0 = pl.multiple_of(b * RB, 8)
    acc = jnp.dot(zbuf[pl.ds(r0, RB), :], p4_ref[0],
                  preferred_element_type=jnp.float32)
    for j in range(1, 4):
        acc += jnp.dot(zbuf[pl.ds(j * VP + r0, RB), :], p4_ref[j],
                       preferred_element_type=jnp.float32)
    o_ref[0] = acc


def _repack(emb_t, eye, p4):
    return pl.pallas_call(
        _repack_body,
        grid=(F, VP // RB),
        in_specs=[
            pl.BlockSpec((D, D), lambda f, b: (0, 0)),
            pl.BlockSpec((4, D, 4 * D), lambda f, b: (0, 0, 0)),
            pl.BlockSpec((1, D, V), lambda f, b: (f, 0, 0),
                         pipeline_mode=pl.Buffered(buffer_count=1)),
        ],
        out_specs=pl.BlockSpec((1, RB, 4 * D), lambda f, b: (f, b, 0)),
        out_shape=jax.ShapeDtypeStruct((F, VP, 4 * D), jnp.float32),
        compiler_params=pltpu.CompilerParams(vmem_limit_bytes=67108864),
        scratch_shapes=[
            pltpu.VMEM((V, D), jnp.float32),
        ],
    )(eye, p4, emb_t)


def _sc_pool(packed, lin2d, idx, idxq, idxo):
    nch = idx.shape[1]
    rows_per_w = nch * CHUNK
    B = NW * rows_per_w
    mesh = plsc.VectorSubcoreMesh(
        core_axis_name="c", subcore_axis_name="s", num_cores=NC, num_subcores=NS
    )

    @functools.partial(
        pl.kernel,
        out_type=[
            jax.ShapeDtypeStruct((D, B), jnp.float32),
            jax.ShapeDtypeStruct((D, B), jnp.float32),
            jax.ShapeDtypeStruct((B,), jnp.float32),
        ],
        mesh=mesh,
        compiler_params=pltpu.CompilerParams(
            use_tc_tiling_on_sc=False, needs_layout_passes=False),
        scratch_types=[
            pltpu.VMEM((F, CHUNK), jnp.int32),
            pltpu.VMEM((F, CHUNK), jnp.int32),
            pltpu.VMEM((F, CHUNK), jnp.int32),
            pltpu.VMEM((F, CHUNK, 4 * D), jnp.float32),
            pltpu.VMEM((F, CHUNK), jnp.float32),
            pltpu.VMEM((D, CHUNK), jnp.float32),
            pltpu.VMEM((D, CHUNK), jnp.float32),
            pltpu.VMEM((CHUNK,), jnp.float32),
            pltpu.SemaphoreType.DMA,
            pltpu.SemaphoreType.DMA,
        ],
    )
    def sc_kernel(packed_hbm, lin_hbm, idx_hbm, idxq_hbm, idxo_hbm,
                  s_hbm, q_hbm, l_hbm,
                  idx_v, qbuf, obuf, ebuf, lbuf, sv, qv, lv, esem, lsem):
        wid = lax.axis_index("s") * NC + lax.axis_index("c")
        base = wid * rows_per_w
        lane = lax.iota(jnp.int32, 16)

        @pl.loop(0, nch)
        def _chunk(c):
            row0 = base + c * CHUNK
            pltpu.sync_copy(idx_hbm.at[wid, c], idx_v)
            pltpu.sync_copy(idxq_hbm.at[wid, c], qbuf)
            pltpu.sync_copy(idxo_hbm.at[wid, c], obuf)

            @pl.loop(0, F)
            def _fire(f):
                pltpu.make_async_copy(
                    packed_hbm.at[qbuf.at[f]], ebuf.at[f], esem).start()
                pltpu.make_async_copy(
                    lin_hbm.at[f].at[idx_v.at[f]], lbuf.at[f], lsem).start()

            @pl.loop(0, F)
            def _drain(f):
                pltpu.make_async_copy(
                    packed_hbm.at[qbuf.at[f]], ebuf.at[f], esem).wait()
                pltpu.make_async_copy(
                    lin_hbm.at[f].at[idx_v.at[f]], lbuf.at[f], lsem).wait()

            for g in range(CHUNK // 16):
                rows16 = g * 16 + lane
                offs = []
                for f in range(F):
                    offs.append(obuf[f, pl.ds(g * 16, 16)])

                @pl.loop(0, D)
                def _dim(d, _g=g, _rows=rows16, _offs=offs):
                    sd = jnp.zeros((16,), jnp.float32)
                    qd = jnp.zeros((16,), jnp.float32)
                    for f in range(F):
                        f16 = jnp.full((16,), f, jnp.int32)
                        x = plsc.load_gather(
                            ebuf, [f16, _rows, _offs[f] + d])
                        sd += x
                        qd += x * x
                    sv[d, pl.ds(_g * 16, 16)] = sd
                    qv[d, pl.ds(_g * 16, 16)] = qd

            @pl.loop(0, CHUNK // 16)
            def _lin(g):
                acc = jnp.zeros((16,), jnp.float32)
                for f in range(F):
                    acc += lbuf[f, pl.ds(g * 16, 16)]
                lv[pl.ds(g * 16, 16)] = acc

            pltpu.sync_copy(sv, s_hbm.at[:, pl.ds(row0, CHUNK)])
            pltpu.sync_copy(qv, q_hbm.at[:, pl.ds(row0, CHUNK)])
            pltpu.sync_copy(lv, l_hbm.at[pl.ds(row0, CHUNK)])

    return sc_kernel(packed, lin2d, idx, idxq, idxo)


def _tc_mlp_body(s_ref, q_ref, l_ref, ds_ref, w1a_ref, w1b_ref, b1_ref,
                 w2_ref, b2_ref, wf_ref, linw_ref, c_ref, o_ref):
    st = s_ref[...]
    qt = q_ref[...]
    ds = ds_ref[...]
    bi_t = 0.5 * (st * st - qt)
    h = lax.dot_general(bi_t, w1a_ref[...], (((0,), (0,)), ((), ())),
                        preferred_element_type=jnp.float32)
    h += jnp.dot(ds, w1b_ref[...], preferred_element_type=jnp.float32)
    h = jnp.maximum(h + b1_ref[...], 0.0)
    h = jnp.dot(h, w2_ref[...], preferred_element_type=jnp.float32)
    h = jnp.maximum(h + b2_ref[...], 0.0)
    z = jnp.dot(h, wf_ref[...], preferred_element_type=jnp.float32)
    z += jnp.dot(ds, linw_ref[...], preferred_element_type=jnp.float32)
    z += l_ref[...] + c_ref[0, 0]
    o_ref[...] = jax.nn.sigmoid(z)


def _tc_mlp(St, Qt, L, ds_input, W1a, W1b, b1, W2, b2, Wf, lin_W, const):
    B = St.shape[1]
    BM = 1024
    grid = (B // BM,)
    DS = ds_input.shape[1]
    H1 = W2.shape[0]
    H2 = W2.shape[1]
    full = lambda shape: pl.BlockSpec(shape, lambda i: (0, 0))
    return pl.pallas_call(
        _tc_mlp_body,
        grid=grid,
        in_specs=[
            pl.BlockSpec((D, BM), lambda i: (0, i)),
            pl.BlockSpec((D, BM), lambda i: (0, i)),
            pl.BlockSpec((BM, 1), lambda i: (i, 0)),
            pl.BlockSpec((BM, DS), lambda i: (i, 0)),
            full((D, H1)),
            full((DS, H1)),
            full((1, H1)),
            full((H1, H2)),
            full((1, H2)),
            full((H2, 1)),
            full((DS, 1)),
            full((1, 1)),
        ],
        out_specs=pl.BlockSpec((BM, 1), lambda i: (i, 0)),
        out_shape=jax.ShapeDtypeStruct((B, 1), jnp.float32),
    )(St, Qt, L, ds_input, W1a, W1b, b1, W2, b2, Wf, lin_W, const)


@jax.jit
def kernel(ds_input, sp_input, emb_tables, lin_tables, lin_W, lin_b,
           W1, b1, W2, b2, Wf, bf):
    B = sp_input.shape[0]
    sp32 = sp_input.astype(jnp.int32)
    nch = B // (NW * CHUNK)
    lay = lambda a: a.reshape(NW, nch, CHUNK, F).transpose(0, 1, 3, 2)
    idx = lay(sp32)
    offs_f = (jnp.arange(F, dtype=jnp.int32) * VP)[None, :]
    idxq = lay(sp32 % VP + offs_f)
    idxo = lay((sp32 // VP) * D)

    emb_t = jnp.transpose(emb_tables, (0, 2, 1))
    eye = jnp.eye(D, dtype=jnp.float32)
    p4 = jnp.stack([jnp.pad(eye, ((0, 0), (j * D, (3 - j) * D)))
                    for j in range(4)])
    packed = _repack(emb_t, eye, p4).reshape(F * VP, 4 * D)

    St, Qt, L = _sc_pool(packed, lin_tables.reshape(F, V), idx, idxq, idxo)

    W1a = W1[:D]
    W1b = W1[D:]
    const = (bf + lin_b).reshape(1, 1)
    out = _tc_mlp(St, Qt, L.reshape(B, 1), ds_input, W1a, W1b,
                  b1.reshape(1, -1), W2, b2.reshape(1, -1), Wf, lin_W, const)
    return out


# speedup vs baseline: 1.2494x; 1.2494x over previous
import functools

import jax
import jax.numpy as jnp
from jax import lax
from jax.experimental import pallas as pl
from jax.experimental.pallas import tpu as pltpu
from jax.experimental.pallas import tpu_sc as plsc

F = 26
V = 100000
D = 32
VP = V // 4
NC = 2
NS = 16
NW = NC * NS
CHUNK = 32
VB = 1000


RB = 1000


def _repack_body(eye_ref, p4_ref, x_ref, o_ref, zbuf):
    b = pl.program_id(1)

    @pl.when(b == 0)
    def _load():
        zbuf[...] = lax.dot_general(
            x_ref[0], eye_ref[...], (((0,), (0,)), ((), ())),
            preferred_element_type=jnp.float32)

    r0 = pl.multiple_of(b * RB, 8)
    acc = jnp.dot(zbuf[pl.ds(r0, RB), :], p4_ref[0],
                  preferred_element_type=jnp.float32)
    for j in range(1, 4):
        acc += jnp.dot(zbuf[pl.ds(j * VP + r0, RB), :], p4_ref[j],
                       preferred_element_type=jnp.float32)
    o_ref[0] = acc


def _repack(emb_t, eye, p4):
    return pl.pallas_call(
        _repack_body,
        grid=(F, VP // RB),
        in_specs=[
            pl.BlockSpec((D, D), lambda f, b: (0, 0)),
            pl.BlockSpec((4, D, 4 * D), lambda f, b: (0, 0, 0)),
            pl.BlockSpec((1, D, V), lambda f, b: (f, 0, 0),
                         pipeline_mode=pl.Buffered(buffer_count=1)),
        ],
        out_specs=pl.BlockSpec((1, RB, 4 * D), lambda f, b: (f, b, 0)),
        out_shape=jax.ShapeDtypeStruct((F, VP, 4 * D), jnp.float32),
        compiler_params=pltpu.CompilerParams(vmem_limit_bytes=67108864),
        scratch_shapes=[
            pltpu.VMEM((V, D), jnp.float32),
        ],
    )(eye, p4, emb_t)


def _sc_pool(packed, lin2d, idx, idxq):
    nch = idx.shape[1]
    rows_per_w = nch * CHUNK
    B = NW * rows_per_w
    mesh = plsc.VectorSubcoreMesh(
        core_axis_name="c", subcore_axis_name="s", num_cores=NC, num_subcores=NS
    )

    @functools.partial(
        pl.kernel,
        out_type=[
            jax.ShapeDtypeStruct((B, D), jnp.float32),
            jax.ShapeDtypeStruct((B, D), jnp.float32),
            jax.ShapeDtypeStruct((B,), jnp.float32),
        ],
        mesh=mesh,
        compiler_params=pltpu.CompilerParams(
            use_tc_tiling_on_sc=False, needs_layout_passes=False),
        scratch_types=[
            pltpu.VMEM((F, CHUNK), jnp.int32),
            pltpu.VMEM((F, CHUNK), jnp.int32),
            pltpu.VMEM((F, CHUNK, D), jnp.float32),
            pltpu.VMEM((F, CHUNK), jnp.float32),
            pltpu.VMEM((CHUNK, D), jnp.float32),
            pltpu.VMEM((CHUNK, D), jnp.float32),
            pltpu.VMEM((CHUNK,), jnp.float32),
            pltpu.SemaphoreType.DMA,
            pltpu.SemaphoreType.DMA,
        ],
    )
    def sc_kernel(packed_hbm, lin_hbm, idx_hbm, idxq_hbm,
                  s_hbm, q_hbm, l_hbm,
                  idx_v, qbuf, ebuf, lbuf, sv, qv, lv, esem, lsem):
        wid = lax.axis_index("s") * NC + lax.axis_index("c")
        base = wid * rows_per_w

        @pl.loop(0, nch)
        def _chunk(c):
            row0 = base + c * CHUNK
            pltpu.sync_copy(idx_hbm.at[wid, c], idx_v)
            pltpu.sync_copy(idxq_hbm.at[wid, c], qbuf)

            @pl.loop(0, F)
            def _fire(f):
                pltpu.make_async_copy(
                    packed_hbm.at[qbuf.at[f]], ebuf.at[f], esem).start()
                pltpu.make_async_copy(
                    lin_hbm.at[f].at[idx_v.at[f]], lbuf.at[f], lsem).start()

            @pl.loop(0, F)
            def _drain(f):
                pltpu.make_async_copy(
                    packed_hbm.at[qbuf.at[f]], ebuf.at[f], esem).wait()
                pltpu.make_async_copy(
                    lin_hbm.at[f].at[idx_v.at[f]], lbuf.at[f], lsem).wait()

            @pl.loop(0, CHUNK)
            def _row(r):
                s0 = jnp.zeros((16,), jnp.float32)
                s1 = jnp.zeros((16,), jnp.float32)
                q0 = jnp.zeros((16,), jnp.float32)
                q1 = jnp.zeros((16,), jnp.float32)
                for f in range(F):
                    x0 = ebuf[f, r, pl.ds(0, 16)]
                    x1 = ebuf[f, r, pl.ds(16, 16)]
                    s0 += x0
                    s1 += x1
                    q0 += x0 * x0
                    q1 += x1 * x1
                sv[r, pl.ds(0, 16)] = s0
                sv[r, pl.ds(16, 16)] = s1
                qv[r, pl.ds(0, 16)] = q0
                qv[r, pl.ds(16, 16)] = q1

            @pl.loop(0, CHUNK // 16)
            def _lin(g):
                acc = jnp.zeros((16,), jnp.float32)
                for f in range(F):
                    acc += lbuf[f, pl.ds(g * 16, 16)]
                lv[pl.ds(g * 16, 16)] = acc

            pltpu.sync_copy(sv, s_hbm.at[pl.ds(row0, CHUNK)])
            pltpu.sync_copy(qv, q_hbm.at[pl.ds(row0, CHUNK)])
            pltpu.sync_copy(lv, l_hbm.at[pl.ds(row0, CHUNK)])

    return sc_kernel(packed, lin2d, idx, idxq)


def _tc_mlp_body(s_ref, q_ref, l_ref, ds_ref, w1a_ref, w1b_ref, b1_ref,
                 w2_ref, b2_ref, wf_ref, linw_ref, c_ref, o_ref):
    s = s_ref[...]
    q = q_ref[...]
    ds = ds_ref[...]
    bi = 0.5 * (s * s - q)
    h = jnp.dot(bi, w1a_ref[...], preferred_element_type=jnp.float32)
    h += jnp.dot(ds, w1b_ref[...], preferred_element_type=jnp.float32)
    h = jnp.maximum(h + b1_ref[...], 0.0)
    h = jnp.dot(h, w2_ref[...], preferred_element_type=jnp.float32)
    h = jnp.maximum(h + b2_ref[...], 0.0)
    z = jnp.dot(h, wf_ref[...], preferred_element_type=jnp.float32)
    z += jnp.dot(ds, linw_ref[...], preferred_element_type=jnp.float32)
    z += l_ref[...] + c_ref[0, 0]
    o_ref[...] = jax.nn.sigmoid(z)


def _tc_mlp(St, Qt, L, ds_input, W1a, W1b, b1, W2, b2, Wf, lin_W, const):
    B = St.shape[0]
    BM = 1024
    grid = (B // BM,)
    DS = ds_input.shape[1]
    H1 = W2.shape[0]
    H2 = W2.shape[1]
    full = lambda shape: pl.BlockSpec(shape, lambda i: (0, 0))
    return pl.pallas_call(
        _tc_mlp_body,
        grid=grid,
        in_specs=[
            pl.BlockSpec((BM, D), lambda i: (i, 0)),
            pl.BlockSpec((BM, D), lambda i: (i, 0)),
            pl.BlockSpec((BM, 1), lambda i: (i, 0)),
            pl.BlockSpec((BM, DS), lambda i: (i, 0)),
            full((D, H1)),
            full((DS, H1)),
            full((1, H1)),
            full((H1, H2)),
            full((1, H2)),
            full((H2, 1)),
            full((DS, 1)),
            full((1, 1)),
        ],
        out_specs=pl.BlockSpec((BM, 1), lambda i: (i, 0)),
        out_shape=jax.ShapeDtypeStruct((B, 1), jnp.float32),
    )(St, Qt, L, ds_input, W1a, W1b, b1, W2, b2, Wf, lin_W, const)


@jax.jit
def kernel(ds_input, sp_input, emb_tables, lin_tables, lin_W, lin_b,
           W1, b1, W2, b2, Wf, bf):
    B = sp_input.shape[0]
    sp32 = sp_input.astype(jnp.int32)
    nch = B // (NW * CHUNK)
    lay = lambda a: a.reshape(NW, nch, CHUNK, F).transpose(0, 1, 3, 2)
    idx = lay(sp32)
    offs_f = (jnp.arange(F, dtype=jnp.int32) * (4 * VP))[None, :]
    idxq = lay(4 * (sp32 % VP) + sp32 // VP + offs_f)

    emb_t = jnp.transpose(emb_tables, (0, 2, 1))
    eye = jnp.eye(D, dtype=jnp.float32)
    p4 = jnp.stack([jnp.pad(eye, ((0, 0), (j * D, (3 - j) * D)))
                    for j in range(4)])
    packed = _repack(emb_t, eye, p4).reshape(F * VP * 4, D)

    St, Qt, L = _sc_pool(packed, lin_tables.reshape(F, V), idx, idxq)

    W1a = W1[:D]
    W1b = W1[D:]
    const = (bf + lin_b).reshape(1, 1)
    out = _tc_mlp(St, Qt, L.reshape(B, 1), ds_input, W1a, W1b,
                  b1.reshape(1, -1), W2, b2.reshape(1, -1), Wf, lin_W, const)
    return out


# speedup vs baseline: 1.2898x; 1.0323x over previous
import functools

import jax
import jax.numpy as jnp
from jax import lax
from jax.experimental import pallas as pl
from jax.experimental.pallas import tpu as pltpu
from jax.experimental.pallas import tpu_sc as plsc

F = 26
V = 100000
D = 32
VP = V // 4
NC = 2
NS = 16
NW = NC * NS
CHUNK = 64
VB = 1000


RB = 1000


def _repack_body(eye_ref, p4_ref, x_ref, o_ref, zbuf):
    b = pl.program_id(1)

    @pl.when(b == 0)
    def _load():
        zbuf[...] = lax.dot_general(
            x_ref[0], eye_ref[...], (((0,), (0,)), ((), ())),
            preferred_element_type=jnp.float32)

    r0 = pl.multiple_of(b * RB, 8)
    acc = jnp.dot(zbuf[pl.ds(r0, RB), :], p4_ref[0],
                  preferred_element_type=jnp.float32)
    for j in range(1, 4):
        acc += jnp.dot(zbuf[pl.ds(j * VP + r0, RB), :], p4_ref[j],
                       preferred_element_type=jnp.float32)
    o_ref[0] = acc


def _repack(emb_t, eye, p4):
    return pl.pallas_call(
        _repack_body,
        grid=(F, VP // RB),
        in_specs=[
            pl.BlockSpec((D, D), lambda f, b: (0, 0)),
            pl.BlockSpec((4, D, 4 * D), lambda f, b: (0, 0, 0)),
            pl.BlockSpec((1, D, V), lambda f, b: (f, 0, 0),
                         pipeline_mode=pl.Buffered(buffer_count=1)),
        ],
        out_specs=pl.BlockSpec((1, RB, 4 * D), lambda f, b: (f, b, 0)),
        out_shape=jax.ShapeDtypeStruct((F, VP, 4 * D), jnp.float32),
        compiler_params=pltpu.CompilerParams(vmem_limit_bytes=67108864),
        scratch_shapes=[
            pltpu.VMEM((V, D), jnp.float32),
        ],
    )(eye, p4, emb_t)


def _sc_pool(packed, lin2d, idx, idxq):
    nch = idx.shape[1]
    rows_per_w = nch * CHUNK
    B = NW * rows_per_w
    mesh = plsc.VectorSubcoreMesh(
        core_axis_name="c", subcore_axis_name="s", num_cores=NC, num_subcores=NS
    )

    @functools.partial(
        pl.kernel,
        out_type=[
            jax.ShapeDtypeStruct((B, D), jnp.float32),
            jax.ShapeDtypeStruct((B, D), jnp.float32),
            jax.ShapeDtypeStruct((B,), jnp.float32),
        ],
        mesh=mesh,
        compiler_params=pltpu.CompilerParams(
            use_tc_tiling_on_sc=False, needs_layout_passes=False),
        scratch_types=[
            pltpu.VMEM((F, CHUNK), jnp.int32),
            pltpu.VMEM((F, CHUNK), jnp.int32),
            pltpu.VMEM((F, CHUNK, D), jnp.float32),
            pltpu.VMEM((F, CHUNK), jnp.float32),
            pltpu.VMEM((CHUNK, D), jnp.float32),
            pltpu.VMEM((CHUNK, D), jnp.float32),
            pltpu.VMEM((CHUNK,), jnp.float32),
            pltpu.SemaphoreType.DMA,
            pltpu.SemaphoreType.DMA,
        ],
    )
    def sc_kernel(packed_hbm, lin_hbm, idx_hbm, idxq_hbm,
                  s_hbm, q_hbm, l_hbm,
                  idx_v, qbuf, ebuf, lbuf, sv, qv, lv, esem, lsem):
        wid = lax.axis_index("s") * NC + lax.axis_index("c")
        base = wid * rows_per_w

        @pl.loop(0, nch)
        def _chunk(c):
            row0 = base + c * CHUNK
            pltpu.sync_copy(idx_hbm.at[wid, c], idx_v)
            pltpu.sync_copy(idxq_hbm.at[wid, c], qbuf)

            @pl.loop(0, F)
            def _fire(f):
                pltpu.make_async_copy(
                    packed_hbm.at[qbuf.at[f]], ebuf.at[f], esem).start()
                pltpu.make_async_copy(
                    lin_hbm.at[f].at[idx_v.at[f]], lbuf.at[f], lsem).start()

            @pl.loop(0, F)
            def _drain(f):
                pltpu.make_async_copy(
                    packed_hbm.at[qbuf.at[f]], ebuf.at[f], esem).wait()
                pltpu.make_async_copy(
                    lin_hbm.at[f].at[idx_v.at[f]], lbuf.at[f], lsem).wait()

            @pl.loop(0, CHUNK)
            def _row(r):
                s0 = jnp.zeros((16,), jnp.float32)
                s1 = jnp.zeros((16,), jnp.float32)
                q0 = jnp.zeros((16,), jnp.float32)
                q1 = jnp.zeros((16,), jnp.float32)
                for f in range(F):
                    x0 = ebuf[f, r, pl.ds(0, 16)]
                    x1 = ebuf[f, r, pl.ds(16, 16)]
                    s0 += x0
                    s1 += x1
                    q0 += x0 * x0
                    q1 += x1 * x1
                sv[r, pl.ds(0, 16)] = s0
                sv[r, pl.ds(16, 16)] = s1
                qv[r, pl.ds(0, 16)] = q0
                qv[r, pl.ds(16, 16)] = q1

            @pl.loop(0, CHUNK // 16)
            def _lin(g):
                acc = jnp.zeros((16,), jnp.float32)
                for f in range(F):
                    acc += lbuf[f, pl.ds(g * 16, 16)]
                lv[pl.ds(g * 16, 16)] = acc

            pltpu.sync_copy(sv, s_hbm.at[pl.ds(row0, CHUNK)])
            pltpu.sync_copy(qv, q_hbm.at[pl.ds(row0, CHUNK)])
            pltpu.sync_copy(lv, l_hbm.at[pl.ds(row0, CHUNK)])

    return sc_kernel(packed, lin2d, idx, idxq)


def _tc_mlp_body(s_ref, q_ref, l_ref, ds_ref, w1a_ref, w1b_ref, b1_ref,
                 w2_ref, b2_ref, wf_ref, linw_ref, c_ref, o_ref):
    s = s_ref[...]
    q = q_ref[...]
    ds = ds_ref[...]
    bi = 0.5 * (s * s - q)
    h = jnp.dot(bi, w1a_ref[...], preferred_element_type=jnp.float32)
    h += jnp.dot(ds, w1b_ref[...], preferred_element_type=jnp.float32)
    h = jnp.maximum(h + b1_ref[...], 0.0)
    h = jnp.dot(h, w2_ref[...], preferred_element_type=jnp.float32)
    h = jnp.maximum(h + b2_ref[...], 0.0)
    z = jnp.dot(h, wf_ref[...], preferred_element_type=jnp.float32)
    z += jnp.dot(ds, linw_ref[...], preferred_element_type=jnp.float32)
    z += l_ref[...] + c_ref[0, 0]
    o_ref[...] = jax.nn.sigmoid(z)


def _tc_mlp(St, Qt, L, ds_input, W1a, W1b, b1, W2, b2, Wf, lin_W, const):
    B = St.shape[0]
    BM = 1024
    grid = (B // BM,)
    DS = ds_input.shape[1]
    H1 = W2.shape[0]
    H2 = W2.shape[1]
    full = lambda shape: pl.BlockSpec(shape, lambda i: (0, 0))
    return pl.pallas_call(
        _tc_mlp_body,
        grid=grid,
        in_specs=[
            pl.BlockSpec((BM, D), lambda i: (i, 0)),
            pl.BlockSpec((BM, D), lambda i: (i, 0)),
            pl.BlockSpec((BM, 1), lambda i: (i, 0)),
            pl.BlockSpec((BM, DS), lambda i: (i, 0)),
            full((D, H1)),
            full((DS, H1)),
            full((1, H1)),
            full((H1, H2)),
            full((1, H2)),
            full((H2, 1)),
            full((DS, 1)),
            full((1, 1)),
        ],
        out_specs=pl.BlockSpec((BM, 1), lambda i: (i, 0)),
        out_shape=jax.ShapeDtypeStruct((B, 1), jnp.float32),
    )(St, Qt, L, ds_input, W1a, W1b, b1, W2, b2, Wf, lin_W, const)


@jax.jit
def kernel(ds_input, sp_input, emb_tables, lin_tables, lin_W, lin_b,
           W1, b1, W2, b2, Wf, bf):
    B = sp_input.shape[0]
    sp32 = sp_input.astype(jnp.int32)
    nch = B // (NW * CHUNK)
    lay = lambda a: a.reshape(NW, nch, CHUNK, F).transpose(0, 1, 3, 2)
    idx = lay(sp32)
    offs_f = (jnp.arange(F, dtype=jnp.int32) * (4 * VP))[None, :]
    idxq = lay(4 * (sp32 % VP) + sp32 // VP + offs_f)

    emb_t = jnp.transpose(emb_tables, (0, 2, 1))
    eye = jnp.eye(D, dtype=jnp.float32)
    p4 = jnp.stack([jnp.pad(eye, ((0, 0), (j * D, (3 - j) * D)))
                    for j in range(4)])
    packed = _repack(emb_t, eye, p4).reshape(F * VP * 4, D)

    St, Qt, L = _sc_pool(packed, lin_tables.reshape(F, V), idx, idxq)

    W1a = W1[:D]
    W1b = W1[D:]
    const = (bf + lin_b).reshape(1, 1)
    out = _tc_mlp(St, Qt, L.reshape(B, 1), ds_input, W1a, W1b,
                  b1.reshape(1, -1), W2, b2.reshape(1, -1), Wf, lin_W, const)
    return out
